# Initial kernel scaffold; baseline (speedup 1.0000x reference)
import jax
import jax.numpy as jnp
from jax.experimental import pallas as pl


def kernel(x, edge_index, W1, att_src1, att_dst1, b1, W2, att_src2, att_dst2, b2):
    raise NotImplementedError("write your pallas kernel here")


# speedup vs baseline: 1.0930x; 1.0930x over previous
import functools

import jax
import jax.numpy as jnp
from jax.experimental import pallas as pl

N = 10000
E = 320000
ROW_BLK = 2000


def _mm_attn_body(x_ref, w_ref, a_ref, h_ref, out_a_ref):
    h = x_ref[...] @ w_ref[...]
    h_ref[...] = h
    out_a_ref[...] = h @ a_ref[...]


def _mm_attn(x, W, A):
    n, f = x.shape
    k = A.shape[1]
    grid = n // ROW_BLK
    return pl.pallas_call(
        _mm_attn_body,
        grid=(grid,),
        in_specs=[
            pl.BlockSpec((ROW_BLK, f), lambda i: (i, 0)),
            pl.BlockSpec((f, 128), lambda i: (0, 0)),
            pl.BlockSpec((128, k), lambda i: (0, 0)),
        ],
        out_specs=[
            pl.BlockSpec((ROW_BLK, 128), lambda i: (i, 0)),
            pl.BlockSpec((ROW_BLK, k), lambda i: (i, 0)),
        ],
        out_shape=[
            jax.ShapeDtypeStruct((n, 128), jnp.float32),
            jax.ShapeDtypeStruct((n, k), jnp.float32),
        ],
    )(x, W, A)


def _elu_mm_attn_body(x_ref, b_ref, w_ref, a_ref, h_ref, out_a_ref):
    xb = x_ref[...] + b_ref[...]
    x = jnp.where(xb > 0, xb, jnp.exp(jnp.minimum(xb, 0.0)) - 1.0)
    h = x @ w_ref[...]
    h_ref[...] = h
    out_a_ref[...] = h @ a_ref[...]


def _elu_mm_attn(x, b, W, A):
    n, f = x.shape
    k = A.shape[1]
    grid = n // ROW_BLK
    return pl.pallas_call(
        _elu_mm_attn_body,
        grid=(grid,),
        in_specs=[
            pl.BlockSpec((ROW_BLK, f), lambda i: (i, 0)),
            pl.BlockSpec((1, f), lambda i: (0, 0)),
            pl.BlockSpec((f, 128), lambda i: (0, 0)),
            pl.BlockSpec((128, k), lambda i: (0, 0)),
        ],
        out_specs=[
            pl.BlockSpec((ROW_BLK, 128), lambda i: (i, 0)),
            pl.BlockSpec((ROW_BLK, k), lambda i: (i, 0)),
        ],
        out_shape=[
            jax.ShapeDtypeStruct((n, 128), jnp.float32),
            jax.ShapeDtypeStruct((n, k), jnp.float32),
        ],
    )(x, b.reshape(1, f), W, A)


def _edge_softmax_agg(h, a_src, a_dst, src, dst, heads, chans):
    n = h.shape[0]
    e = a_src[src] + a_dst[dst]
    e = jnp.where(e > 0, e, 0.2 * e)
    p = jnp.exp(e)
    denom = jax.ops.segment_sum(p, dst, num_segments=n)
    alpha = p / (denom[dst] + 1e-16)
    hr = h.reshape(n, heads, chans)
    msg = hr[src] * alpha[:, :, None]
    out = jax.ops.segment_sum(msg, dst, num_segments=n)
    return out.reshape(n, heads * chans)


def kernel(x, edge_index, W1, att_src1, att_dst1, b1, W2, att_src2, att_dst2, b2):
    n = x.shape[0]
    loop = jnp.arange(n, dtype=edge_index.dtype)
    src = jnp.concatenate([edge_index[0], loop])
    dst = jnp.concatenate([edge_index[1], loop])

    H1, C1 = att_src1.shape
    eye1 = jnp.eye(H1, dtype=x.dtype)
    As1 = (eye1[:, None, :] * att_src1[:, :, None]).reshape(H1 * C1, H1)
    Ad1 = (eye1[:, None, :] * att_dst1[:, :, None]).reshape(H1 * C1, H1)
    A1 = jnp.concatenate([As1, Ad1], axis=1)

    h, a1 = _mm_attn(x, W1, A1)
    out1 = _edge_softmax_agg(h, a1[:, :H1], a1[:, H1:], src, dst, H1, C1)

    A2 = jnp.concatenate([att_src2.T, att_dst2.T], axis=1)
    g, a2 = _elu_mm_attn(out1, b1, W2, A2)
    out2 = _edge_softmax_agg(g, a2[:, :1], a2[:, 1:], src, dst, 1, 128)
    return out2 + b2


# speedup vs baseline: 28.1937x; 25.7954x over previous
import functools

import jax
import jax.numpy as jnp
from jax import lax
from jax.experimental import pallas as pl
from jax.experimental.pallas import tpu as pltpu
from jax.experimental.pallas import tpu_sc as plsc

N = 10000
E = 320000
NP = 10240
CH = 128
NCH = 81
NTILES = 32
EP = NTILES * CH * NCH
ROW_BLK = 2000
ACCW = 136


def _permute16(x, idx):
    dnums = lax.GatherDimensionNumbers(
        offset_dims=(), collapsed_slice_dims=(0,), start_index_map=(0,))
    return lax.gather(x, idx[:, None], dnums, (1,),
                      mode=lax.GatherScatterMode.PROMISE_IN_BOUNDS)


def _make_edge_kernel(heads):
    mesh = plsc.VectorSubcoreMesh(core_axis_name="c", subcore_axis_name="s")
    rows_per_sub = NP // 16

    @functools.partial(
        pl.kernel,
        mesh=mesh,
        compiler_params=pltpu.CompilerParams(use_tc_tiling_on_sc=False),
        out_type=jax.ShapeDtypeStruct((2, NP, ACCW), jnp.float32),
        scratch_types=[
            pltpu.VMEM((CH,), jnp.int32),
            pltpu.VMEM((CH,), jnp.int32),
            pltpu.VMEM((CH, 16), jnp.float32),
            pltpu.VMEM((CH, 16), jnp.float32),
            pltpu.VMEM((CH, 128), jnp.float32),
            pltpu.VMEM((CH, ACCW), jnp.float32),
            pltpu.VMEM((8, ACCW), jnp.float32),
            pltpu.VMEM_SHARED((NP, ACCW), jnp.float32),
            pltpu.SemaphoreType.DMA,
            pltpu.SemaphoreType.DMA,
            pltpu.SemaphoreType.DMA,
        ],
    )
    def edge_kernel(a_hbm, h_hbm, src_hbm, dst_hbm, out_hbm,
                    src_v, dst_v, u_v, v_v, h_v, m_v, z_v, acc_sh,
                    s0, s1, s2):
        cid = lax.axis_index("c")
        sid = lax.axis_index("s")
        wid = cid * 16 + sid
        lane = lax.iota(jnp.int32, 16)
        zero16 = jnp.zeros((16,), jnp.float32)

        for r in range(8):
            for cc in range(8):
                z_v[r, pl.ds(cc * 16, 16)] = zero16
            z_v[r, pl.ds(ACCW - 16, 16)] = zero16
        row0 = sid * rows_per_sub
        for kk in range(rows_per_sub // 8):
            pltpu.sync_copy(z_v, acc_sh.at[pl.ds(row0 + kk * 8, 8)])
        plsc.subcore_barrier()

        rot = 8 + (lane & 7)
        pmask = lane < heads

        def chunk_body(g, carry):
            base = (wid * NCH + g) * CH
            pltpu.sync_copy(src_hbm.at[pl.ds(base, CH)], src_v)
            pltpu.sync_copy(dst_hbm.at[pl.ds(base, CH)], dst_v)
            cp0 = pltpu.async_copy(a_hbm.at[src_v], u_v, s0)
            cp1 = pltpu.async_copy(a_hbm.at[dst_v], v_v, s1)
            cp2 = pltpu.async_copy(h_hbm.at[src_v], h_v, s2)
            cp0.wait()
            cp1.wait()
            cp2.wait()

            def edge_body(j, carry2):
                u = u_v[j]
                v = v_v[j]
                if heads == 8:
                    e = u + _permute16(v, rot)
                else:
                    e = (_permute16(u, lane * 0)
                         + _permute16(v, lane * 0 + 1))
                e = jnp.maximum(e, 0.2 * e)
                p = jnp.exp(e)
                m_v[j, pl.ds(0, 16)] = jnp.where(pmask, p, 0.0)
                for jh in range(8):
                    if heads == 8:
                        pj = _permute16(p, lane * 0 + jh)
                    else:
                        pj = p
                    m_v[j, pl.ds(8 + jh * 16, 16)] = h_v[j, pl.ds(jh * 16, 16)] * pj
                return carry2

            lax.fori_loop(0, CH, edge_body, 0)
            pltpu.sync_copy(m_v, acc_sh.at[dst_v], add=True)
            return carry

        lax.fori_loop(0, NCH, chunk_body, 0)
        plsc.subcore_barrier()
        pltpu.sync_copy(acc_sh.at[pl.ds(row0, rows_per_sub)],
                        out_hbm.at[cid, pl.ds(row0, rows_per_sub)])

    return edge_kernel


_edge_sc_8 = _make_edge_kernel(8)
_edge_sc_1 = _make_edge_kernel(1)


def _mm_attn_body(x_ref, w_ref, a_ref, h_ref, out_a_ref):
    h = x_ref[...] @ w_ref[...]
    h_ref[...] = h
    out_a_ref[...] = h @ a_ref[...]


def _mm_attn(x, W, A):
    n, f = x.shape
    k = A.shape[1]
    return pl.pallas_call(
        _mm_attn_body,
        grid=(n // ROW_BLK,),
        in_specs=[
            pl.BlockSpec((ROW_BLK, f), lambda i: (i, 0)),
            pl.BlockSpec((f, 128), lambda i: (0, 0)),
            pl.BlockSpec((128, k), lambda i: (0, 0)),
        ],
        out_specs=[
            pl.BlockSpec((ROW_BLK, 128), lambda i: (i, 0)),
            pl.BlockSpec((ROW_BLK, k), lambda i: (i, 0)),
        ],
        out_shape=[
            jax.ShapeDtypeStruct((n, 128), jnp.float32),
            jax.ShapeDtypeStruct((n, k), jnp.float32),
        ],
    )(x, W, A)


def _norm_elu_mm_body(s_ref, r_ref, b_ref, w_ref, a_ref, g_ref, a2_ref):
    acc = s_ref[0] + s_ref[1]
    msg = acc[:, 8:136]
    den = acc[:, 0:8]
    dex = den @ r_ref[...]
    out1 = msg / (dex + 1e-16)
    xb = out1 + b_ref[...]
    h1 = jnp.where(xb > 0, xb, jnp.exp(jnp.minimum(xb, 0.0)) - 1.0)
    g = h1 @ w_ref[...]
    g_ref[...] = g
    a2_ref[...] = g @ a_ref[...]


def _norm_elu_mm(s, R816, b, W, A):
    n = s.shape[1]
    k = A.shape[1]
    return pl.pallas_call(
        _norm_elu_mm_body,
        grid=(n // ROW_BLK,),
        in_specs=[
            pl.BlockSpec((2, ROW_BLK, ACCW), lambda i: (0, i, 0)),
            pl.BlockSpec((8, 128), lambda i: (0, 0)),
            pl.BlockSpec((1, 128), lambda i: (0, 0)),
            pl.BlockSpec((128, 128), lambda i: (0, 0)),
            pl.BlockSpec((128, k), lambda i: (0, 0)),
        ],
        out_specs=[
            pl.BlockSpec((ROW_BLK, 128), lambda i: (i, 0)),
            pl.BlockSpec((ROW_BLK, k), lambda i: (i, 0)),
        ],
        out_shape=[
            jax.ShapeDtypeStruct((n, 128), jnp.float32),
            jax.ShapeDtypeStruct((n, k), jnp.float32),
        ],
    )(s, R816, b.reshape(1, 128), W, A)


def _final_body(s_ref, b_ref, o_ref):
    acc = s_ref[0] + s_ref[1]
    msg = acc[:, 8:136]
    den = acc[:, 0:1]
    o_ref[...] = msg / (den + 1e-16) + b_ref[...]


def _final(s, b):
    n = s.shape[1]
    return pl.pallas_call(
        _final_body,
        grid=(n // ROW_BLK,),
        in_specs=[
            pl.BlockSpec((2, ROW_BLK, ACCW), lambda i: (0, i, 0)),
            pl.BlockSpec((1, 128), lambda i: (0, 0)),
        ],
        out_specs=pl.BlockSpec((ROW_BLK, 128), lambda i: (i, 0)),
        out_shape=jax.ShapeDtypeStruct((n, 128), jnp.float32),
    )(s, b.reshape(1, 128))


def kernel(x, edge_index, W1, att_src1, att_dst1, b1, W2, att_src2, att_dst2, b2):
    n = x.shape[0]
    loop = jnp.arange(n, dtype=edge_index.dtype)
    pad = EP - (E + n)
    src_all = jnp.concatenate(
        [edge_index[0], loop, jnp.full((pad,), n, edge_index.dtype)])
    dst_all = jnp.concatenate(
        [edge_index[1], loop, jnp.full((pad,), n, edge_index.dtype)])

    H1, C1 = att_src1.shape
    eye1 = jnp.eye(H1, dtype=x.dtype)
    As1 = (eye1[:, None, :] * att_src1[:, :, None]).reshape(H1 * C1, H1)
    Ad1 = (eye1[:, None, :] * att_dst1[:, :, None]).reshape(H1 * C1, H1)
    A1 = jnp.concatenate([As1, Ad1], axis=1)
    A2 = jnp.concatenate([att_src2.T, att_dst2.T], axis=1)
    R816 = jnp.repeat(jnp.eye(8, dtype=x.dtype), 16, axis=1)

    h, a1 = _mm_attn(x, W1, A1)
    h_p = jnp.pad(h, ((0, NP - n), (0, 0)))
    a1_p = jnp.pad(a1, ((0, NP - n), (0, 0)))
    s1 = _edge_sc_8(a1_p, h_p, src_all, dst_all)

    g, a2 = _norm_elu_mm(s1[:, :n, :], R816, b1, W2, A2)
    g_p = jnp.pad(g, ((0, NP - n), (0, 0)))
    a2_p = jnp.pad(a2, ((0, NP - n), (0, 14)))
    s2 = _edge_sc_1(a2_p, g_p, src_all, dst_all)

    return _final(s2[:, :n, :], b2)


# speedup vs baseline: 64.1486x; 2.2753x over previous
import functools

import jax
import jax.numpy as jnp
from jax import lax
from jax.experimental import pallas as pl
from jax.experimental.pallas import tpu as pltpu
from jax.experimental.pallas import tpu_sc as plsc

N = 10000
E = 320000
NP = 10240
CH = 64
NCH = 162
NTILES = 32
EP = NTILES * CH * NCH
ROW_BLK = 2000
ACCW = 136


def _permute16(x, idx):
    dnums = lax.GatherDimensionNumbers(
        offset_dims=(), collapsed_slice_dims=(0,), start_index_map=(0,))
    return lax.gather(x, idx[:, None], dnums, (1,),
                      mode=lax.GatherScatterMode.PROMISE_IN_BOUNDS)


def _make_edge_kernel(heads):
    mesh = plsc.VectorSubcoreMesh(core_axis_name="c", subcore_axis_name="s")
    rows_per_sub = NP // 16

    buf_scratch = [
        pltpu.VMEM((CH,), jnp.int32),
        pltpu.VMEM((CH,), jnp.int32),
        pltpu.VMEM((CH, 16), jnp.float32),
        pltpu.VMEM((CH, 16), jnp.float32),
        pltpu.VMEM((CH, 128), jnp.float32),
        pltpu.VMEM((CH, ACCW), jnp.float32),
        pltpu.VMEM((CH,), jnp.int32),
    ]

    @functools.partial(
        pl.kernel,
        mesh=mesh,
        compiler_params=pltpu.CompilerParams(use_tc_tiling_on_sc=False),
        out_type=jax.ShapeDtypeStruct((2, NP, ACCW), jnp.float32),
        scratch_types=buf_scratch + buf_scratch + [
            pltpu.VMEM((8, ACCW), jnp.float32),
            pltpu.VMEM_SHARED((NP, ACCW), jnp.float32),
            pltpu.SemaphoreType.DMA,
            pltpu.SemaphoreType.DMA,
            pltpu.SemaphoreType.DMA,
            pltpu.SemaphoreType.DMA,
        ],
    )
    def edge_kernel(a_hbm, h_hbm, src_hbm, dst_hbm, out_hbm,
                    sv0, dv0, u0, v0, hh0, m0, w0,
                    sv1, dv1, u1, v1, hh1, m1, w1,
                    z_v, acc_sh, g0, g1, q0, q1):
        bufs = ((sv0, dv0, u0, v0, hh0, m0, w0, g0, q0),
                (sv1, dv1, u1, v1, hh1, m1, w1, g1, q1))
        cid = lax.axis_index("c")
        sid = lax.axis_index("s")
        wid = cid * 16 + sid
        lane = lax.iota(jnp.int32, 16)
        zero16 = jnp.zeros((16,), jnp.float32)

        for r in range(8):
            for cc in range(8):
                z_v[r, pl.ds(cc * 16, 16)] = zero16
            z_v[r, pl.ds(ACCW - 16, 16)] = zero16
        row0 = sid * rows_per_sub
        for kk in range(rows_per_sub // 8):
            pltpu.sync_copy(z_v, acc_sh.at[pl.ds(row0 + kk * 8, 8)])
        plsc.subcore_barrier()

        rot = 8 + (lane & 7)
        pmask = lane < heads

        def issue_idx(c, S):
            sv, dv = S[0], S[1]
            base = (wid * NCH + c) * CH
            pltpu.async_copy(src_hbm.at[pl.ds(base, CH)], sv, S[7])
            pltpu.async_copy(dst_hbm.at[pl.ds(base, CH)], dv, S[7])

        def issue_gathers(c, S):
            sv, dv, u_v, v_v, h_v = S[0], S[1], S[2], S[3], S[4]
            base = (wid * NCH + c) * CH
            pltpu.make_async_copy(src_hbm.at[pl.ds(base, CH)], sv, S[7]).wait()
            pltpu.make_async_copy(dst_hbm.at[pl.ds(base, CH)], dv, S[7]).wait()
            pltpu.async_copy(a_hbm.at[sv], u_v, S[7])
            pltpu.async_copy(a_hbm.at[dv], v_v, S[7])
            pltpu.async_copy(h_hbm.at[sv], h_v, S[7])

        def wait_gathers(S):
            sv, dv, u_v, v_v, h_v = S[0], S[1], S[2], S[3], S[4]
            pltpu.make_async_copy(a_hbm.at[sv], u_v, S[7]).wait()
            pltpu.make_async_copy(a_hbm.at[dv], v_v, S[7]).wait()
            pltpu.make_async_copy(h_hbm.at[sv], h_v, S[7]).wait()

        def wait_scatter(S):
            pltpu.make_async_copy(S[5], acc_sh.at[S[6]], S[8]).wait()

        def compute_scatter(S):
            dv, u_v, v_v, h_v, m_v, w_v = S[1], S[2], S[3], S[4], S[5], S[6]
            for k in range(CH // 16):
                w_v[pl.ds(k * 16, 16)] = dv[pl.ds(k * 16, 16)]

            @plsc.parallel_loop(0, CH, unroll=2)
            def edge_body(j):
                u = u_v[j]
                v = v_v[j]
                if heads == 8:
                    e = u + _permute16(v, rot)
                else:
                    e = (_permute16(u, lane * 0)
                         + _permute16(v, lane * 0 + 1))
                e = jnp.maximum(e, 0.2 * e)
                p = jnp.exp(e)
                m_v[j, pl.ds(0, 16)] = jnp.where(pmask, p, 0.0)
                for jh in range(8):
                    if heads == 8:
                        pj = _permute16(p, lane * 0 + jh)
                    else:
                        pj = p
                    m_v[j, pl.ds(8 + jh * 16, 16)] = h_v[j, pl.ds(jh * 16, 16)] * pj

            pltpu.async_copy(m_v, acc_sh.at[w_v], S[8], add=True)

        def chunk(c, X, Y, first_pair, last_pair):
            wait_gathers(X)
            if not last_pair:
                issue_idx(c + 1, Y)
            if not first_pair:
                wait_scatter(X)
            if not last_pair:
                issue_gathers(c + 1, Y)
            compute_scatter(X)

        issue_idx(0, bufs[0])
        issue_gathers(0, bufs[0])
        chunk(0, bufs[0], bufs[1], True, False)
        chunk(1, bufs[1], bufs[0], True, False)

        def pair_body(t, carry):
            chunk(2 * t, bufs[0], bufs[1], False, False)
            chunk(2 * t + 1, bufs[1], bufs[0], False, False)
            return carry

        lax.fori_loop(1, NCH // 2 - 1, pair_body, 0)
        chunk(NCH - 2, bufs[0], bufs[1], False, False)
        chunk(NCH - 1, bufs[1], bufs[0], False, True)
        wait_scatter(bufs[0])
        wait_scatter(bufs[1])

        plsc.subcore_barrier()
        pltpu.sync_copy(acc_sh.at[pl.ds(row0, rows_per_sub)],
                        out_hbm.at[cid, pl.ds(row0, rows_per_sub)])

    return edge_kernel


_edge_sc_8 = _make_edge_kernel(8)
_edge_sc_1 = _make_edge_kernel(1)


def _mm_attn_body(x_ref, w_ref, a_ref, h_ref, out_a_ref):
    h = x_ref[...] @ w_ref[...]
    h_ref[...] = h
    out_a_ref[...] = h @ a_ref[...]


def _mm_attn(x, W, A):
    n, f = x.shape
    k = A.shape[1]
    return pl.pallas_call(
        _mm_attn_body,
        grid=(n // ROW_BLK,),
        in_specs=[
            pl.BlockSpec((ROW_BLK, f), lambda i: (i, 0)),
            pl.BlockSpec((f, 128), lambda i: (0, 0)),
            pl.BlockSpec((128, k), lambda i: (0, 0)),
        ],
        out_specs=[
            pl.BlockSpec((ROW_BLK, 128), lambda i: (i, 0)),
            pl.BlockSpec((ROW_BLK, k), lambda i: (i, 0)),
        ],
        out_shape=[
            jax.ShapeDtypeStruct((n, 128), jnp.float32),
            jax.ShapeDtypeStruct((n, k), jnp.float32),
        ],
    )(x, W, A)


def _norm_elu_mm_body(s_ref, r_ref, b_ref, w_ref, a_ref, g_ref, a2_ref):
    acc = s_ref[0] + s_ref[1]
    msg = acc[:, 8:136]
    den = acc[:, 0:8]
    dex = den @ r_ref[...]
    out1 = msg / (dex + 1e-16)
    xb = out1 + b_ref[...]
    h1 = jnp.where(xb > 0, xb, jnp.exp(jnp.minimum(xb, 0.0)) - 1.0)
    g = h1 @ w_ref[...]
    g_ref[...] = g
    a2_ref[...] = g @ a_ref[...]


def _norm_elu_mm(s, R816, b, W, A):
    n = s.shape[1]
    k = A.shape[1]
    return pl.pallas_call(
        _norm_elu_mm_body,
        grid=(n // ROW_BLK,),
        in_specs=[
            pl.BlockSpec((2, ROW_BLK, ACCW), lambda i: (0, i, 0)),
            pl.BlockSpec((8, 128), lambda i: (0, 0)),
            pl.BlockSpec((1, 128), lambda i: (0, 0)),
            pl.BlockSpec((128, 128), lambda i: (0, 0)),
            pl.BlockSpec((128, k), lambda i: (0, 0)),
        ],
        out_specs=[
            pl.BlockSpec((ROW_BLK, 128), lambda i: (i, 0)),
            pl.BlockSpec((ROW_BLK, k), lambda i: (i, 0)),
        ],
        out_shape=[
            jax.ShapeDtypeStruct((n, 128), jnp.float32),
            jax.ShapeDtypeStruct((n, k), jnp.float32),
        ],
    )(s, R816, b.reshape(1, 128), W, A)


def _final_body(s_ref, b_ref, o_ref):
    acc = s_ref[0] + s_ref[1]
    msg = acc[:, 8:136]
    den = acc[:, 0:1]
    o_ref[...] = msg / (den + 1e-16) + b_ref[...]


def _final(s, b):
    n = s.shape[1]
    return pl.pallas_call(
        _final_body,
        grid=(n // ROW_BLK,),
        in_specs=[
            pl.BlockSpec((2, ROW_BLK, ACCW), lambda i: (0, i, 0)),
            pl.BlockSpec((1, 128), lambda i: (0, 0)),
        ],
        out_specs=pl.BlockSpec((ROW_BLK, 128), lambda i: (i, 0)),
        out_shape=jax.ShapeDtypeStruct((n, 128), jnp.float32),
    )(s, b.reshape(1, 128))


def kernel(x, edge_index, W1, att_src1, att_dst1, b1, W2, att_src2, att_dst2, b2):
    n = x.shape[0]
    loop = jnp.arange(n, dtype=edge_index.dtype)
    pad = EP - (E + n)
    src_all = jnp.concatenate(
        [edge_index[0], loop, jnp.full((pad,), n, edge_index.dtype)])
    dst_all = jnp.concatenate(
        [edge_index[1], loop, jnp.full((pad,), n, edge_index.dtype)])

    H1, C1 = att_src1.shape
    eye1 = jnp.eye(H1, dtype=x.dtype)
    As1 = (eye1[:, None, :] * att_src1[:, :, None]).reshape(H1 * C1, H1)
    Ad1 = (eye1[:, None, :] * att_dst1[:, :, None]).reshape(H1 * C1, H1)
    A1 = jnp.concatenate([As1, Ad1], axis=1)
    A2 = jnp.concatenate([att_src2.T, att_dst2.T], axis=1)
    R816 = jnp.repeat(jnp.eye(8, dtype=x.dtype), 16, axis=1)

    h, a1 = _mm_attn(x, W1, A1)
    h_p = jnp.pad(h, ((0, NP - n), (0, 0)))
    a1_p = jnp.pad(a1, ((0, NP - n), (0, 0)))
    s1 = _edge_sc_8(a1_p, h_p, src_all, dst_all)

    g, a2 = _norm_elu_mm(s1[:, :n, :], R816, b1, W2, A2)
    g_p = jnp.pad(g, ((0, NP - n), (0, 0)))
    a2_p = jnp.pad(a2, ((0, NP - n), (0, 14)))
    s2 = _edge_sc_1(a2_p, g_p, src_all, dst_all)

    return _final(s2[:, :n, :], b2)
